# baseline (speedup 1.0000x reference)
import functools
import jax
import jax.numpy as jnp
from jax import lax
from jax.experimental import pallas as pl
from jax.experimental.pallas import tpu as pltpu
from jax.experimental.pallas import tpu_sc as plsc

NPTS = 4194304
RGRID = 513
NPATCH = RGRID * RGRID
CROWS = 2104
NW = 32
PPW = NPTS // NW
CHUNK = 2048
NCH = PPW // CHUNK
NG = CHUNK // 16
SLEN = 128
NSTR = CHUNK // SLEN
XROWS = NPTS * 2 // 128
OROWS = NPTS // 128
RSUB = 11
CBROWS = 58


def _spline_body(x3, cf2, out, p16, xv, f0b, f1b, idxb, patches, outb,
                 cbuf, pbuf, gsem0, gsem1, osem0, osem1, xsem0, xsem1):
    cid = lax.axis_index("c")
    sid = lax.axis_index("s")
    wid = sid * 2 + cid
    roff = cid * NPATCH
    lane = lax.iota(jnp.int32, 16)
    patc = (lane >> 2) * 516 + (lane & 3)

    for s in range(3):
        rb = sid * 33 + s * RSUB

        @pl.when(rb < RGRID)
        def _():
            fstart = rb * 516
            sr = fstart >> 7
            rel0 = fstart - sr * 128
            pltpu.sync_copy(cf2.at[pl.ds(sr, CBROWS), :], cbuf)

            def per_r(t, _):
                r = rb + t

                @pl.when(r < RGRID)
                def _():
                    relr = rel0 + t * 516

                    @plsc.parallel_loop(0, RGRID, unroll=4)
                    def per_c(c):
                        rel = relr + c + patc
                        pbuf[c, :] = plsc.load_gather(
                            cbuf, [rel >> 7, rel & 127])

                    pltpu.sync_copy(
                        pbuf, p16.at[pl.ds(roff + r * RGRID, RGRID), :])

                return 0

            lax.fori_loop(0, RSUB, per_r, 0)

    plsc.subcore_barrier()

    def phase_a(g1, bs, gsem, xsem):
        pltpu.make_async_copy(
            x3.at[pl.ds(0, CHUNK // 64), :],
            xv.at[pl.ds(bs * (CHUNK // 64), CHUNK // 64), :],
            xsem,
        ).wait()

        @plsc.parallel_loop(0, NSTR)
        def stream_body(j):
            xrow = bs * (CHUNK // 64) + 2 * j

            def idx_body(k, _):
                i = j * 8 + k
                x0 = xv[xrow, pl.ds(k * 16, 16)]
                x1 = xv[xrow + 1, pl.ds(k * 16, 16)]
                t0 = x0 * 512.0 + 0.5
                t1 = x1 * 512.0 + 0.5
                r0 = t0.astype(jnp.int32)
                r1 = t1.astype(jnp.int32)
                f0 = t0 - r0.astype(jnp.float32)
                f1 = t1 - r1.astype(jnp.float32)
                f0b[pl.ds(bs * CHUNK + i * 16, 16)] = f0
                f1b[pl.ds(bs * CHUNK + i * 16, 16)] = f1
                idxb[pl.ds(bs * CHUNK + i * 16, 16)] = r0 * RGRID + r1 + roff
                return 0

            lax.fori_loop(0, 8, idx_body, 0, unroll=True)
            pltpu.async_copy(
                p16.at[idxb.at[pl.ds(bs * CHUNK + j * SLEN, SLEN)]],
                patches.at[pl.ds(bs * CHUNK + j * SLEN, SLEN), :],
                gsem,
            )
        @pl.when(g1 + 2 < NCH)
        def _():
            base2 = wid * PPW + (g1 + 2) * CHUNK
            pltpu.async_copy(
                x3.at[pl.ds(base2 // 64, CHUNK // 64), :],
                xv.at[pl.ds(bs * (CHUNK // 64), CHUNK // 64), :],
                xsem,
            )

    def gather_drain(gsem):
        pltpu.make_async_copy(
            p16.at[pl.ds(0, CHUNK), :],
            patches.at[pl.ds(0, CHUNK), :],
            gsem,
        ).wait()

    def out_drain(osem):
        pltpu.make_async_copy(
            outb.at[pl.ds(0, CHUNK // 128), :],
            out.at[pl.ds(0, CHUNK // 128), :],
            osem,
        ).wait()

    base0 = wid * PPW

    pltpu.async_copy(
        x3.at[pl.ds(base0 // 64, CHUNK // 64), :],
        xv.at[pl.ds(0, CHUNK // 64), :], xsem0)
    pltpu.async_copy(
        x3.at[pl.ds(base0 // 64 + CHUNK // 64, CHUNK // 64), :],
        xv.at[pl.ds(CHUNK // 64, CHUNK // 64), :], xsem1)
    phase_a(0, 0, gsem0, xsem0)

    def body(g, _):
        nxt = g + 1
        odd_n = (nxt & 1) == 1
        odd_g = (g & 1) == 1

        @pl.when(jnp.logical_and(nxt < NCH, jnp.logical_not(odd_n)))
        def _():
            phase_a(nxt, 0, gsem0, xsem0)

        @pl.when(jnp.logical_and(nxt < NCH, odd_n))
        def _():
            phase_a(nxt, 1, gsem1, xsem1)

        @pl.when(jnp.logical_not(odd_g))
        def _():
            gather_drain(gsem0)

        @pl.when(odd_g)
        def _():
            gather_drain(gsem1)

        @pl.when(jnp.logical_and(g >= 2, jnp.logical_not(odd_g)))
        def _():
            out_drain(osem0)

        @pl.when(jnp.logical_and(g >= 2, odd_g))
        def _():
            out_drain(osem1)

        bofs = (g & 1) * CHUNK

        @plsc.parallel_loop(0, NG, unroll=3)
        def out_body(i):
            f0 = f0b[pl.ds(bofs + i * 16, 16)]
            f1 = f1b[pl.ds(bofs + i * 16, 16)]
            u0 = 1.0 - f0
            u1 = 1.0 - f1
            f0sq = f0 * f0
            f1sq = f1 * f1
            u0sq = u0 * u0
            u1sq = u1 * u1
            b1 = (u0sq * u0, (3.0 * f0 - 6.0) * f0sq + 4.0,
                  (3.0 * u0 - 6.0) * u0sq + 4.0, f0sq * f0)
            b2 = (u1sq * u1, (3.0 * f1 - 6.0) * f1sq + 4.0,
                  (3.0 * u1 - 6.0) * u1sq + 4.0, f1sq * f1)
            row = bofs + i * 16 + lane
            acc = jnp.zeros((16,), jnp.float32)
            for j1 in range(4):
                col = jnp.full((16,), j1 * 4, jnp.int32)
                racc = plsc.load_gather(patches, [row, col]) * b2[0]
                for j2 in range(1, 4):
                    col = jnp.full((16,), j1 * 4 + j2, jnp.int32)
                    racc = racc + plsc.load_gather(patches, [row, col]) * b2[j2]
                acc = acc + racc * b1[j1]
            outb[(g & 1) * 16 + (i >> 3), pl.ds((i & 7) * 16, 16)] = acc
        base = wid * PPW + g * CHUNK

        @pl.when(jnp.logical_not(odd_g))
        def _():
            pltpu.async_copy(
                outb.at[pl.ds(0, CHUNK // 128), :],
                out.at[pl.ds(base // 128, CHUNK // 128), :],
                osem0,
            )

        @pl.when(odd_g)
        def _():
            pltpu.async_copy(
                outb.at[pl.ds(16, CHUNK // 128), :],
                out.at[pl.ds(base // 128, CHUNK // 128), :],
                osem1,
            )

        return 0

    lax.fori_loop(0, NCH, body, 0)
    out_drain(osem0)
    out_drain(osem1)


@jax.jit
def _run(x3, cf2):
    mesh = plsc.VectorSubcoreMesh(core_axis_name="c", subcore_axis_name="s")
    f = pl.kernel(
        _spline_body,
        out_type=jax.ShapeDtypeStruct((OROWS, 128), jnp.float32),
        mesh=mesh,
        scratch_types=[
            pltpu.HBM((2 * NPATCH, 16), jnp.float32),
            pltpu.VMEM((2 * CHUNK // 64, 128), jnp.float32),
            pltpu.VMEM((2 * CHUNK,), jnp.float32),
            pltpu.VMEM((2 * CHUNK,), jnp.float32),
            pltpu.VMEM((2 * CHUNK,), jnp.int32),
            pltpu.VMEM((2 * CHUNK, 16), jnp.float32),
            pltpu.VMEM((2 * CHUNK // 128, 128), jnp.float32),
            pltpu.VMEM((CBROWS, 128), jnp.float32),
            pltpu.VMEM((RGRID, 16), jnp.float32),
            pltpu.SemaphoreType.DMA,
            pltpu.SemaphoreType.DMA,
            pltpu.SemaphoreType.DMA,
            pltpu.SemaphoreType.DMA,
            pltpu.SemaphoreType.DMA,
            pltpu.SemaphoreType.DMA,
        ],
        compiler_params=pltpu.CompilerParams(
            needs_layout_passes=False, use_tc_tiling_on_sc=False),
    )
    return f(x3, cf2)


def kernel(x, coeffs):
    x3 = x.reshape(32768, 128, 2).transpose(0, 2, 1).reshape(XROWS, 128)
    cf2 = jnp.pad(coeffs.reshape(-1), (0, CROWS * 128 - 516 * 516))
    cf2 = cf2.reshape(CROWS, 128)
    out = _run(x3, cf2)
    return out.reshape(NPTS, 1)


# speedup vs baseline: 1.0128x; 1.0128x over previous
import functools
import jax
import jax.numpy as jnp
from jax import lax
from jax.experimental import pallas as pl
from jax.experimental.pallas import tpu as pltpu
from jax.experimental.pallas import tpu_sc as plsc

NPTS = 4194304
RGRID = 513
NPATCH = RGRID * RGRID
CROWS = 2104
NW = 32
PPW = NPTS // NW
CHUNK = 2048
NCH = PPW // CHUNK
NG = CHUNK // 16
SLEN = 128
NSTR = CHUNK // SLEN
XROWS = NPTS * 2 // 128
OROWS = NPTS // 128
RSUB = 11
CBROWS = 58


def _spline_body(x3, cf2, out, p16, xv, f0b, f1b, idxb, patches, outb,
                 cbuf, pbuf, gsem0, gsem1, osem0, osem1, xsem0, xsem1):
    cid = lax.axis_index("c")
    sid = lax.axis_index("s")
    wid = sid * 2 + cid
    roff = cid * NPATCH
    lane = lax.iota(jnp.int32, 16)
    patc = (lane >> 2) * 516 + (lane & 3)

    for s in range(3):
        rb = sid * 33 + s * RSUB

        @pl.when(rb < RGRID)
        def _():
            fstart = rb * 516
            sr = fstart >> 7
            rel0 = fstart - sr * 128
            pltpu.sync_copy(cf2.at[pl.ds(sr, CBROWS), :], cbuf)

            def per_r(t, _):
                r = rb + t

                @pl.when(r < RGRID)
                def _():
                    relr = rel0 + t * 516

                    @plsc.parallel_loop(0, RGRID, unroll=4)
                    def per_c(c):
                        rel = relr + c + patc
                        pbuf[c, :] = plsc.load_gather(
                            cbuf, [rel >> 7, rel & 127])

                    pltpu.sync_copy(
                        pbuf, p16.at[pl.ds(roff + r * RGRID, RGRID), :])

                return 0

            lax.fori_loop(0, RSUB, per_r, 0)

    plsc.subcore_barrier()

    def phase_a(g1, bs, gsem, xsem):
        pltpu.make_async_copy(
            x3.at[pl.ds(0, CHUNK // 64), :],
            xv.at[pl.ds(bs * (CHUNK // 64), CHUNK // 64), :],
            xsem,
        ).wait()

        @plsc.parallel_loop(0, NSTR)
        def stream_body(j):
            xrow = bs * (CHUNK // 64) + 2 * j

            def idx_body(k, _):
                i = j * 8 + k
                x0 = xv[xrow, pl.ds(k * 16, 16)]
                x1 = xv[xrow + 1, pl.ds(k * 16, 16)]
                t0 = x0 * 512.0 + 0.5
                t1 = x1 * 512.0 + 0.5
                r0 = t0.astype(jnp.int32)
                r1 = t1.astype(jnp.int32)
                f0 = t0 - r0.astype(jnp.float32)
                f1 = t1 - r1.astype(jnp.float32)
                f0b[pl.ds(bs * CHUNK + i * 16, 16)] = f0
                f1b[pl.ds(bs * CHUNK + i * 16, 16)] = f1
                idxb[pl.ds(bs * CHUNK + i * 16, 16)] = r0 * RGRID + r1 + roff
                return 0

            lax.fori_loop(0, 8, idx_body, 0, unroll=True)
            pltpu.async_copy(
                p16.at[idxb.at[pl.ds(bs * CHUNK + j * SLEN, SLEN)]],
                patches.at[pl.ds(bs * CHUNK + j * SLEN, SLEN), :],
                gsem,
            )
        @pl.when(g1 + 2 < NCH)
        def _():
            base2 = wid * PPW + (g1 + 2) * CHUNK
            pltpu.async_copy(
                x3.at[pl.ds(base2 // 64, CHUNK // 64), :],
                xv.at[pl.ds(bs * (CHUNK // 64), CHUNK // 64), :],
                xsem,
            )

    def gather_drain(gsem):
        pltpu.make_async_copy(
            p16.at[pl.ds(0, CHUNK), :],
            patches.at[pl.ds(0, CHUNK), :],
            gsem,
        ).wait()

    def out_drain(osem):
        pltpu.make_async_copy(
            outb.at[pl.ds(0, CHUNK // 128), :],
            out.at[pl.ds(0, CHUNK // 128), :],
            osem,
        ).wait()

    base0 = wid * PPW

    pltpu.async_copy(
        x3.at[pl.ds(base0 // 64, CHUNK // 64), :],
        xv.at[pl.ds(0, CHUNK // 64), :], xsem0)
    pltpu.async_copy(
        x3.at[pl.ds(base0 // 64 + CHUNK // 64, CHUNK // 64), :],
        xv.at[pl.ds(CHUNK // 64, CHUNK // 64), :], xsem1)
    phase_a(0, 0, gsem0, xsem0)

    def body(g, _):
        nxt = g + 1
        odd_n = (nxt & 1) == 1
        odd_g = (g & 1) == 1

        @pl.when(jnp.logical_and(nxt < NCH, jnp.logical_not(odd_n)))
        def _():
            phase_a(nxt, 0, gsem0, xsem0)

        @pl.when(jnp.logical_and(nxt < NCH, odd_n))
        def _():
            phase_a(nxt, 1, gsem1, xsem1)

        @pl.when(jnp.logical_not(odd_g))
        def _():
            gather_drain(gsem0)

        @pl.when(odd_g)
        def _():
            gather_drain(gsem1)

        @pl.when(jnp.logical_and(g >= 2, jnp.logical_not(odd_g)))
        def _():
            out_drain(osem0)

        @pl.when(jnp.logical_and(g >= 2, odd_g))
        def _():
            out_drain(osem1)

        bofs = (g & 1) * CHUNK

        @plsc.parallel_loop(0, NG, unroll=2)
        def out_body(i):
            f0 = f0b[pl.ds(bofs + i * 16, 16)]
            f1 = f1b[pl.ds(bofs + i * 16, 16)]
            u0 = 1.0 - f0
            u1 = 1.0 - f1
            f0sq = f0 * f0
            f1sq = f1 * f1
            u0sq = u0 * u0
            u1sq = u1 * u1
            b1 = (u0sq * u0, (3.0 * f0 - 6.0) * f0sq + 4.0,
                  (3.0 * u0 - 6.0) * u0sq + 4.0, f0sq * f0)
            b2 = (u1sq * u1, (3.0 * f1 - 6.0) * f1sq + 4.0,
                  (3.0 * u1 - 6.0) * u1sq + 4.0, f1sq * f1)
            row = bofs + i * 16 + lane
            acc = jnp.zeros((16,), jnp.float32)
            for j1 in range(4):
                col = jnp.full((16,), j1 * 4, jnp.int32)
                racc = plsc.load_gather(patches, [row, col]) * b2[0]
                for j2 in range(1, 4):
                    col = jnp.full((16,), j1 * 4 + j2, jnp.int32)
                    racc = racc + plsc.load_gather(patches, [row, col]) * b2[j2]
                acc = acc + racc * b1[j1]
            outb[(g & 1) * 16 + (i >> 3), pl.ds((i & 7) * 16, 16)] = acc
        base = wid * PPW + g * CHUNK

        @pl.when(jnp.logical_not(odd_g))
        def _():
            pltpu.async_copy(
                outb.at[pl.ds(0, CHUNK // 128), :],
                out.at[pl.ds(base // 128, CHUNK // 128), :],
                osem0,
            )

        @pl.when(odd_g)
        def _():
            pltpu.async_copy(
                outb.at[pl.ds(16, CHUNK // 128), :],
                out.at[pl.ds(base // 128, CHUNK // 128), :],
                osem1,
            )

        return 0

    lax.fori_loop(0, NCH, body, 0)
    out_drain(osem0)
    out_drain(osem1)


@jax.jit
def _run(x3, cf2):
    mesh = plsc.VectorSubcoreMesh(core_axis_name="c", subcore_axis_name="s")
    f = pl.kernel(
        _spline_body,
        out_type=jax.ShapeDtypeStruct((OROWS, 128), jnp.float32),
        mesh=mesh,
        scratch_types=[
            pltpu.HBM((2 * NPATCH, 16), jnp.float32),
            pltpu.VMEM((2 * CHUNK // 64, 128), jnp.float32),
            pltpu.VMEM((2 * CHUNK,), jnp.float32),
            pltpu.VMEM((2 * CHUNK,), jnp.float32),
            pltpu.VMEM((2 * CHUNK,), jnp.int32),
            pltpu.VMEM((2 * CHUNK, 16), jnp.float32),
            pltpu.VMEM((2 * CHUNK // 128, 128), jnp.float32),
            pltpu.VMEM((CBROWS, 128), jnp.float32),
            pltpu.VMEM((RGRID, 16), jnp.float32),
            pltpu.SemaphoreType.DMA,
            pltpu.SemaphoreType.DMA,
            pltpu.SemaphoreType.DMA,
            pltpu.SemaphoreType.DMA,
            pltpu.SemaphoreType.DMA,
            pltpu.SemaphoreType.DMA,
        ],
        compiler_params=pltpu.CompilerParams(
            needs_layout_passes=False, use_tc_tiling_on_sc=False),
    )
    return f(x3, cf2)


def kernel(x, coeffs):
    x3 = x.reshape(32768, 128, 2).transpose(0, 2, 1).reshape(XROWS, 128)
    cf2 = jnp.pad(coeffs.reshape(-1), (0, CROWS * 128 - 516 * 516))
    cf2 = cf2.reshape(CROWS, 128)
    out = _run(x3, cf2)
    return out.reshape(NPTS, 1)
